# baseline (speedup 1.0000x reference)
import functools

import jax
import jax.numpy as jnp
from jax import lax
from jax.experimental import pallas as pl
from jax.experimental.pallas import tpu as pltpu
from jax.experimental.pallas import tpu_sc as plsc

GAMMA = 12.0
HIDDEN = 128
BATCH = 4096

_info = plsc.get_sparse_core_info()
_NC, _NS = _info.num_cores, _info.num_subcores
_NW = _NC * _NS
_BPW = BATCH // _NW
_HALF = _BPW // 2


def _make_kernel():
    mesh = plsc.VectorSubcoreMesh(core_axis_name="c", subcore_axis_name="s")

    @functools.partial(
        pl.kernel,
        mesh=mesh,
        out_type=jax.ShapeDtypeStruct((BATCH,), jnp.float32),
        scratch_types=[
            pltpu.VMEM((_BPW,), jnp.int32),
            pltpu.VMEM((_BPW,), jnp.int32),
            pltpu.VMEM((_BPW,), jnp.int32),
            pltpu.VMEM((_HALF, HIDDEN), jnp.float32),
            pltpu.VMEM((_HALF, HIDDEN), jnp.float32),
            pltpu.VMEM((_HALF, HIDDEN), jnp.float32),
            pltpu.VMEM((_HALF, HIDDEN), jnp.float32),
            pltpu.VMEM((_BPW,), jnp.float32),
            pltpu.VMEM((16 * 48,), jnp.float32),
            pltpu.SemaphoreType.DMA,
            pltpu.SemaphoreType.DMA,
            pltpu.SemaphoreType.DMA,
            pltpu.SemaphoreType.DMA,
            pltpu.SemaphoreType.DMA,
            pltpu.SemaphoreType.DMA,
        ],
    )
    def transe(ent_hbm, rel_hbm, heads_hbm, rels_hbm, tails_hbm, out_hbm,
               hidx, ridx, tidx, h0buf, h1buf, t0buf, t1buf, scores, w,
               s0, s1, s2, s3, s4, s5):
        wid = lax.axis_index("s") * _NC + lax.axis_index("c")
        base = wid * _BPW

        ci_h = pltpu.async_copy(heads_hbm.at[pl.ds(base, _BPW)], hidx, s0)
        ci_r = pltpu.async_copy(rels_hbm.at[pl.ds(base, _BPW)], ridx, s1)
        ci_t = pltpu.async_copy(tails_hbm.at[pl.ds(base, _BPW)], tidx, s2)
        ci_h.wait()
        ci_r.wait()
        ci_t.wait()

        lo, hi = pl.ds(0, _HALF), pl.ds(_HALF, _HALF)
        cp_h0 = pltpu.async_copy(ent_hbm.at[hidx.at[lo]], h0buf, s0)
        cp_t0 = pltpu.async_copy(ent_hbm.at[tidx.at[lo]], t0buf, s1)
        cp_h1 = pltpu.async_copy(ent_hbm.at[hidx.at[hi]], h1buf, s2)
        cp_t1 = pltpu.async_copy(ent_hbm.at[tidx.at[hi]], t1buf, s3)
        cp_h0.wait()
        cp_r0 = pltpu.async_copy(rel_hbm.at[ridx.at[lo]], h0buf, s4, add=True)
        cp_r0.wait()
        cp_t0.wait()
        cp_h1.wait()
        cp_r1 = pltpu.async_copy(rel_hbm.at[ridx.at[hi]], h1buf, s5, add=True)

        lane = lax.iota(jnp.int32, 16)

        def make_gbody(hbuf, tbuf, off):
            def gbody(g, _):
                res = jnp.zeros((16,), jnp.float32)
                for j in range(16):
                    b = g * 16 + j
                    acc = jnp.zeros((16,), jnp.float32)
                    for c in range(HIDDEN // 16):
                        hv = hbuf[b, pl.ds(c * 16, 16)]
                        tv = tbuf[b, pl.ds(c * 16, 16)]
                        acc = acc + jnp.abs(hv - tv)
                    p = j * 48 + 16
                    w[pl.ds(p, 16)] = acc
                    r1 = acc + w[pl.ds(p + 8, 16)]
                    w[pl.ds(p, 16)] = r1
                    r2 = r1 + w[pl.ds(p + 4, 16)]
                    w[pl.ds(p, 16)] = r2
                    r3 = r2 + w[pl.ds(p + 2, 16)]
                    w[pl.ds(p, 16)] = r3
                    r4 = r3 + w[pl.ds(p + 1, 16)]
                    w[pl.ds(p, 16)] = r4
                    f = w[pl.ds(p - j, 16)]
                    res = jnp.where(lane == j, f, res)
                scores[pl.ds(off + g * 16, 16)] = GAMMA - res
                return 0
            return gbody

        lax.fori_loop(0, _HALF // 16, make_gbody(h0buf, t0buf, 0), 0)
        cp_r1.wait()
        cp_t1.wait()
        lax.fori_loop(0, _HALF // 16, make_gbody(h1buf, t1buf, _HALF), 0)

        pltpu.sync_copy(scores, out_hbm.at[pl.ds(base, _BPW)])

    return transe


_transe = _make_kernel()


def kernel(sample, entity_embedding, relation_embedding):
    heads = sample[:, 0]
    rels = sample[:, 1]
    tails = sample[:, 2]
    scores = _transe(entity_embedding, relation_embedding, heads, rels, tails)
    return scores[:, None]


# speedup vs baseline: 1.0724x; 1.0724x over previous
import functools

import jax
import jax.numpy as jnp
from jax import lax
from jax.experimental import pallas as pl
from jax.experimental.pallas import tpu as pltpu
from jax.experimental.pallas import tpu_sc as plsc

GAMMA = 12.0
HIDDEN = 128
BATCH = 4096

_info = plsc.get_sparse_core_info()
_NC, _NS = _info.num_cores, _info.num_subcores
_NW = _NC * _NS
_BPW = BATCH // _NW
_HALF = _BPW // 2


def _make_kernel():
    mesh = plsc.VectorSubcoreMesh(core_axis_name="c", subcore_axis_name="s")

    @functools.partial(
        pl.kernel,
        mesh=mesh,
        out_type=jax.ShapeDtypeStruct((BATCH,), jnp.float32),
        scratch_types=[
            pltpu.VMEM((_BPW,), jnp.int32),
            pltpu.VMEM((_BPW,), jnp.int32),
            pltpu.VMEM((_BPW,), jnp.int32),
            pltpu.VMEM((_BPW, HIDDEN), jnp.float32),
            pltpu.VMEM((_BPW, HIDDEN), jnp.float32),
            pltpu.VMEM((_BPW,), jnp.float32),
            pltpu.VMEM((16 * 48,), jnp.float32),
            pltpu.SemaphoreType.DMA,
            pltpu.SemaphoreType.DMA,
            pltpu.SemaphoreType.DMA,
        ],
    )
    def transe(ent_hbm, rel_hbm, heads_hbm, rels_hbm, tails_hbm, out_hbm,
               hidx, ridx, tidx, hbuf, tbuf, scores, w, sem_h, sem_r, sem_t):
        wid = lax.axis_index("s") * _NC + lax.axis_index("c")
        base = wid * _BPW

        ci_h = pltpu.async_copy(heads_hbm.at[pl.ds(base, _BPW)], hidx, sem_h)
        ci_r = pltpu.async_copy(rels_hbm.at[pl.ds(base, _BPW)], ridx, sem_r)
        ci_t = pltpu.async_copy(tails_hbm.at[pl.ds(base, _BPW)], tidx, sem_t)
        ci_h.wait()
        ci_r.wait()
        ci_t.wait()

        cp_h = pltpu.async_copy(ent_hbm.at[hidx], hbuf, sem_h)
        cp_t = pltpu.async_copy(ent_hbm.at[tidx], tbuf, sem_t)
        cp_h.wait()
        cp_r = pltpu.async_copy(rel_hbm.at[ridx], hbuf, sem_r, add=True)
        cp_r.wait()
        cp_t.wait()

        lane = lax.iota(jnp.int32, 16)

        def gbody(g, _):
            def jbody(q, res):
                for u in range(4):
                    j = q * 4 + u
                    b = g * 16 + j
                    acc = jnp.zeros((16,), jnp.float32)
                    for c in range(HIDDEN // 16):
                        hv = hbuf[b, pl.ds(c * 16, 16)]
                        tv = tbuf[b, pl.ds(c * 16, 16)]
                        acc = acc + jnp.abs(hv - tv)
                    p = j * 48 + 16
                    w[pl.ds(p, 16)] = acc
                    r1 = acc + w[pl.ds(p + 8, 16)]
                    w[pl.ds(p, 16)] = r1
                    r2 = r1 + w[pl.ds(p + 4, 16)]
                    w[pl.ds(p, 16)] = r2
                    r3 = r2 + w[pl.ds(p + 2, 16)]
                    w[pl.ds(p, 16)] = r3
                    r4 = r3 + w[pl.ds(p + 1, 16)]
                    w[pl.ds(p, 16)] = r4
                    f = w[pl.ds(p - j, 16)]
                    res = jnp.where(lane == j, f, res)
                return res

            res = lax.fori_loop(0, 4, jbody, jnp.zeros((16,), jnp.float32))
            scores[pl.ds(g * 16, 16)] = GAMMA - res
            return 0

        lax.fori_loop(0, _BPW // 16, gbody, 0)

        pltpu.sync_copy(scores, out_hbm.at[pl.ds(base, _BPW)])

    return transe


_transe = _make_kernel()


def kernel(sample, entity_embedding, relation_embedding):
    heads = sample[:, 0]
    rels = sample[:, 1]
    tails = sample[:, 2]
    scores = _transe(entity_embedding, relation_embedding, heads, rels, tails)
    return scores[:, None]


# speedup vs baseline: 1.0770x; 1.0042x over previous
import functools

import jax
import jax.numpy as jnp
from jax import lax
from jax.experimental import pallas as pl
from jax.experimental.pallas import tpu as pltpu
from jax.experimental.pallas import tpu_sc as plsc

GAMMA = 12.0
HIDDEN = 128
BATCH = 4096

_info = plsc.get_sparse_core_info()
_NC, _NS = _info.num_cores, _info.num_subcores
_NW = _NC * _NS
_BPW = BATCH // _NW
_HALF = _BPW // 2


def _make_kernel():
    mesh = plsc.VectorSubcoreMesh(core_axis_name="c", subcore_axis_name="s")

    @functools.partial(
        pl.kernel,
        mesh=mesh,
        out_type=jax.ShapeDtypeStruct((BATCH,), jnp.float32),
        scratch_types=[
            pltpu.VMEM((_BPW,), jnp.int32),
            pltpu.VMEM((_BPW,), jnp.int32),
            pltpu.VMEM((_BPW,), jnp.int32),
            pltpu.VMEM((_BPW, HIDDEN), jnp.float32),
            pltpu.VMEM((_BPW, HIDDEN), jnp.float32),
            pltpu.VMEM((_BPW,), jnp.float32),
            pltpu.VMEM((16 * 48,), jnp.float32),
            pltpu.SemaphoreType.DMA,
            pltpu.SemaphoreType.DMA,
            pltpu.SemaphoreType.DMA,
        ],
    )
    def transe(ent_hbm, rel_hbm, heads_hbm, rels_hbm, tails_hbm, out_hbm,
               hidx, ridx, tidx, hbuf, tbuf, scores, w, sem_h, sem_r, sem_t):
        wid = lax.axis_index("s") * _NC + lax.axis_index("c")
        base = wid * _BPW

        ci_h = pltpu.async_copy(heads_hbm.at[pl.ds(base, _BPW)], hidx, sem_h)
        ci_r = pltpu.async_copy(rels_hbm.at[pl.ds(base, _BPW)], ridx, sem_r)
        ci_t = pltpu.async_copy(tails_hbm.at[pl.ds(base, _BPW)], tidx, sem_t)
        ci_h.wait()
        ci_r.wait()
        ci_t.wait()

        cp_h = pltpu.async_copy(ent_hbm.at[hidx], hbuf, sem_h)
        cp_t = pltpu.async_copy(ent_hbm.at[tidx], tbuf, sem_t)
        cp_h.wait()
        cp_r = pltpu.async_copy(rel_hbm.at[ridx], hbuf, sem_r, add=True)
        cp_r.wait()
        cp_t.wait()

        lane = lax.iota(jnp.int32, 16)

        def gbody(g, _):
            def jbody(q, res):
                for u in range(2):
                    j = q * 2 + u
                    b = g * 16 + j
                    acc = jnp.zeros((16,), jnp.float32)
                    for c in range(HIDDEN // 16):
                        hv = hbuf[b, pl.ds(c * 16, 16)]
                        tv = tbuf[b, pl.ds(c * 16, 16)]
                        acc = acc + jnp.abs(hv - tv)
                    p = j * 48 + 16
                    w[pl.ds(p, 16)] = acc
                    r1 = acc + w[pl.ds(p + 8, 16)]
                    w[pl.ds(p, 16)] = r1
                    r2 = r1 + w[pl.ds(p + 4, 16)]
                    w[pl.ds(p, 16)] = r2
                    r3 = r2 + w[pl.ds(p + 2, 16)]
                    w[pl.ds(p, 16)] = r3
                    r4 = r3 + w[pl.ds(p + 1, 16)]
                    w[pl.ds(p, 16)] = r4
                    f = w[pl.ds(p - j, 16)]
                    res = jnp.where(lane == j, f, res)
                return res

            res = lax.fori_loop(0, 8, jbody, jnp.zeros((16,), jnp.float32))
            scores[pl.ds(g * 16, 16)] = GAMMA - res
            return 0

        lax.fori_loop(0, _BPW // 16, gbody, 0)

        pltpu.sync_copy(scores, out_hbm.at[pl.ds(base, _BPW)])

    return transe


_transe = _make_kernel()


def kernel(sample, entity_embedding, relation_embedding):
    heads = sample[:, 0]
    rels = sample[:, 1]
    tails = sample[:, 2]
    scores = _transe(entity_embedding, relation_embedding, heads, rels, tails)
    return scores[:, None]
